# Initial kernel scaffold; baseline (speedup 1.0000x reference)
import jax
import jax.numpy as jnp
from jax.experimental import pallas as pl


def kernel(module_input, gate_input, W0, b0, W_rest, b_rest, G0_w, G0_b, G1_w, G1_b):
    raise NotImplementedError("write your pallas kernel here")


# speedup vs baseline: 5.1295x; 5.1295x over previous
import functools

import jax
import jax.numpy as jnp
from jax.experimental import pallas as pl

_MODULE_NUM = 8
_OFFS = [0, 1, 3, 6, 10, 15, 21, 28, 36]
_GOUT = 36


def _dt(a, b):
    return jax.lax.dot_general(
        a, b, (((1,), (1,)), ((), ())), preferred_element_type=jnp.float32
    )


def _routing(gl):
    blk = gl.shape[0]
    col = jax.lax.broadcasted_iota(jnp.int32, (1, _GOUT), 1)
    neg = jnp.float32(-1e30)
    gates = jnp.zeros_like(gl)
    onehot = jnp.zeros_like(gl)
    softm = jnp.zeros_like(gl)
    for i in range(_MODULE_NUM):
        lo, hi = _OFFS[i], _OFFS[i + 1]
        m = hi - lo
        mask = jnp.logical_and(col >= lo, col < hi)
        maskf = mask.astype(jnp.float32)
        if m == 1:
            gates = gates + maskf
            onehot = onehot + maskf
            softm = softm + maskf
            continue
        glm = jnp.where(mask, gl, neg)
        m1 = jnp.max(glm, axis=-1, keepdims=True)
        is1 = glm == m1
        idx1 = jnp.min(jnp.where(is1, col, _GOUT * 2), axis=-1, keepdims=True)
        oh1 = (col == idx1).astype(jnp.float32)
        glm2 = jnp.where(col == idx1, neg, glm)
        m2 = jnp.max(glm2, axis=-1, keepdims=True)
        is2 = glm2 == m2
        idx2 = jnp.min(jnp.where(is2, col, _GOUT * 2), axis=-1, keepdims=True)
        oh2 = (col == idx2).astype(jnp.float32)
        p1 = jax.nn.sigmoid(m1 - m2)
        gates = gates + oh1 * p1 + oh2 * (1.0 - p1)
        onehot = onehot + oh1 + oh2
        e = jnp.where(mask, jnp.exp(glm - m1), 0.0)
        s = jnp.sum(e, axis=-1, keepdims=True)
        softm = softm + e / s
    return gates, onehot, softm


def _kern(x_ref, gi_ref, w0_ref, b0_ref, wr_ref, br_ref, g0_ref, g0b_ref,
          g1_ref, g1b_ref, out_ref, gates_ref, oh_ref, sm_ref):
    gi = gi_ref[...]
    h = jnp.maximum(_dt(gi, g0_ref[...]) + g0b_ref[...], 0.0)
    gl = _dt(h, g1_ref[...]) + g1b_ref[...]
    gates, onehot, softm = _routing(gl)
    gates_ref[...] = gates
    oh_ref[...] = onehot
    sm_ref[...] = softm

    x = x_ref[...]
    outs = [jnp.maximum(_dt(x, w0_ref[...]) + b0_ref[...], 0.0)]
    for i in range(_MODULE_NUM - 1):
        fc_in = outs[0] * gates[:, _OFFS[i]:_OFFS[i] + 1]
        for j in range(1, i + 1):
            fc_in = fc_in + outs[j] * gates[:, _OFFS[i] + j:_OFFS[i] + j + 1]
        fc = jnp.maximum(_dt(fc_in, wr_ref[i]) + br_ref[i], 0.0) + fc_in
        outs.append(fc)
    last = outs[0] * gates[:, _OFFS[7]:_OFFS[7] + 1]
    for j in range(1, _MODULE_NUM):
        last = last + outs[j] * gates[:, _OFFS[7] + j:_OFFS[7] + j + 1]
    out_ref[...] = last


@functools.partial(jax.jit, static_argnames=())
def kernel(module_input, gate_input, W0, b0, W_rest, b_rest, G0_w, G0_b, G1_w, G1_b):
    B, D = module_input.shape
    H = W0.shape[0]
    blk = 256
    grid = (B // blk,)

    def row_map(i):
        return (i, 0)

    def const_map2(i):
        return (0, 0)

    def const_map1(i):
        return (0,)

    def const_map3(i):
        return (0, 0, 0)

    out_shapes = (
        jax.ShapeDtypeStruct((B, H), jnp.float32),
        jax.ShapeDtypeStruct((B, _GOUT), jnp.float32),
        jax.ShapeDtypeStruct((B, _GOUT), jnp.float32),
        jax.ShapeDtypeStruct((B, _GOUT), jnp.float32),
    )
    in_specs = [
        pl.BlockSpec((blk, D), row_map),
        pl.BlockSpec((blk, gate_input.shape[1]), row_map),
        pl.BlockSpec(W0.shape, const_map2),
        pl.BlockSpec(b0.shape, const_map1),
        pl.BlockSpec(W_rest.shape, const_map3),
        pl.BlockSpec(b_rest.shape, const_map2),
        pl.BlockSpec(G0_w.shape, const_map2),
        pl.BlockSpec(G0_b.shape, const_map1),
        pl.BlockSpec(G1_w.shape, const_map2),
        pl.BlockSpec(G1_b.shape, const_map1),
    ]
    out_specs = (
        pl.BlockSpec((blk, H), row_map),
        pl.BlockSpec((blk, _GOUT), row_map),
        pl.BlockSpec((blk, _GOUT), row_map),
        pl.BlockSpec((blk, _GOUT), row_map),
    )
    return pl.pallas_call(
        _kern,
        grid=grid,
        in_specs=in_specs,
        out_specs=out_specs,
        out_shape=out_shapes,
    )(module_input, gate_input, W0, b0, W_rest, b_rest, G0_w, G0_b, G1_w, G1_b)
